# baseline (device time: 29798 ns/iter reference)
import jax
import jax.numpy as jnp
from jax import lax
from jax.experimental import pallas as pl
from jax.experimental.pallas import tpu as pltpu

N_DEV = 8
B = 2
SQ = 256
HALO = 128
KV_BUF = SQ + 2 * HALO
HQ = 4
DH = 64
DM = 512
SKV_GLOBAL = N_DEV * SQ


def kernel(x, Wq, K_ext, V_ext, Wo):
    Kt = jnp.transpose(K_ext, (0, 2, 1, 3))
    Vt = jnp.transpose(V_ext, (0, 2, 1, 3))
    Wqt = jnp.transpose(Wq.reshape(DM, HQ, DH), (1, 0, 2)) * 0.125

    def body(x_ref, wqt_ref, k_ref, v_ref, wo_ref, out_ref,
             kbuf, vbuf, send_sems, recv_sems):
        my = lax.axis_index("i")
        left = jnp.maximum(my - 1, 0)
        right = jnp.minimum(my + 1, N_DEV - 1)

        kbuf[:, :, HALO:HALO + SQ] = k_ref[...]
        vbuf[:, :, HALO:HALO + SQ] = v_ref[...]

        @pl.when(my == 0)
        def _():
            kbuf[:, :, 0:HALO] = jnp.zeros((B, HQ, HALO, DH), jnp.float32)
            vbuf[:, :, 0:HALO] = jnp.zeros((B, HQ, HALO, DH), jnp.float32)

        @pl.when(my == N_DEV - 1)
        def _():
            kbuf[:, :, HALO + SQ:] = jnp.zeros((B, HQ, HALO, DH), jnp.float32)
            vbuf[:, :, HALO + SQ:] = jnp.zeros((B, HQ, HALO, DH), jnp.float32)

        rdma_r_k = pltpu.make_async_remote_copy(
            src_ref=k_ref.at[:, :, pl.ds(SQ - HALO, HALO)],
            dst_ref=kbuf.at[:, :, pl.ds(0, HALO)],
            send_sem=send_sems.at[0], recv_sem=recv_sems.at[0],
            device_id=(right,), device_id_type=pltpu.DeviceIdType.MESH,
        )
        rdma_r_v = pltpu.make_async_remote_copy(
            src_ref=v_ref.at[:, :, pl.ds(SQ - HALO, HALO)],
            dst_ref=vbuf.at[:, :, pl.ds(0, HALO)],
            send_sem=send_sems.at[1], recv_sem=recv_sems.at[1],
            device_id=(right,), device_id_type=pltpu.DeviceIdType.MESH,
        )
        rdma_l_k = pltpu.make_async_remote_copy(
            src_ref=k_ref.at[:, :, pl.ds(0, HALO)],
            dst_ref=kbuf.at[:, :, pl.ds(HALO + SQ, HALO)],
            send_sem=send_sems.at[2], recv_sem=recv_sems.at[2],
            device_id=(left,), device_id_type=pltpu.DeviceIdType.MESH,
        )
        rdma_l_v = pltpu.make_async_remote_copy(
            src_ref=v_ref.at[:, :, pl.ds(0, HALO)],
            dst_ref=vbuf.at[:, :, pl.ds(HALO + SQ, HALO)],
            send_sem=send_sems.at[3], recv_sem=recv_sems.at[3],
            device_id=(left,), device_id_type=pltpu.DeviceIdType.MESH,
        )

        @pl.when(my < N_DEV - 1)
        def _():
            rdma_r_k.start()
            rdma_r_v.start()

        @pl.when(my > 0)
        def _():
            rdma_l_k.start()
            rdma_l_v.start()

        q = [
            [jnp.dot(x_ref[b], wqt_ref[h],
                     preferred_element_type=jnp.float32)
             for h in range(HQ)]
            for b in range(B)
        ]

        @pl.when(my > 0)
        def _():
            rdma_r_k.wait_recv()
            rdma_r_v.wait_recv()

        @pl.when(my < N_DEV - 1)
        def _():
            rdma_l_k.wait_recv()
            rdma_l_v.wait_recv()

        @pl.when(my < N_DEV - 1)
        def _():
            rdma_r_k.wait_send()
            rdma_r_v.wait_send()

        @pl.when(my > 0)
        def _():
            rdma_l_k.wait_send()
            rdma_l_v.wait_send()

        r_idx = lax.broadcasted_iota(jnp.int32, (SQ, KV_BUF), 0)
        j_idx = lax.broadcasted_iota(jnp.int32, (SQ, KV_BUF), 1)
        d = j_idx - r_idx
        kglob = my * SQ - HALO + j_idx
        mask = (d >= 0) & (d <= 2 * HALO) & (kglob >= 0) & (kglob < SKV_GLOBAL)

        for b in range(B):
            acc = jnp.zeros((SQ, DM), jnp.float32)
            for h in range(HQ):
                kh = kbuf[b, h]
                vh = vbuf[b, h]
                s = lax.dot_general(
                    q[b][h], kh, (((1,), (1,)), ((), ())),
                    preferred_element_type=jnp.float32,
                )
                s = jnp.where(mask, s, -1e9)
                m = jnp.max(s, axis=1, keepdims=True)
                w = jnp.exp(s - m)
                w = w / jnp.sum(w, axis=1, keepdims=True)
                ctx_h = jnp.dot(w, vh, preferred_element_type=jnp.float32)
                acc += jnp.dot(ctx_h, wo_ref[h * DH:(h + 1) * DH, :],
                               preferred_element_type=jnp.float32)
            out_ref[b] = acc

    return pl.pallas_call(
        body,
        out_shape=jax.ShapeDtypeStruct(x.shape, jnp.float32),
        in_specs=[pl.BlockSpec(memory_space=pltpu.VMEM)] * 5,
        out_specs=pl.BlockSpec(memory_space=pltpu.VMEM),
        scratch_shapes=[
            pltpu.VMEM((B, HQ, KV_BUF, DH), jnp.float32),
            pltpu.VMEM((B, HQ, KV_BUF, DH), jnp.float32),
            pltpu.SemaphoreType.DMA((4,)),
            pltpu.SemaphoreType.DMA((4,)),
        ],
    )(x, Wqt, Kt, Vt, Wo)


# device time: 28742 ns/iter; 1.0367x vs baseline; 1.0367x over previous
import jax
import jax.numpy as jnp
from jax import lax
from jax.experimental import pallas as pl
from jax.experimental.pallas import tpu as pltpu

N_DEV = 8
B = 2
SQ = 256
HALO = 128
HQ = 4
DH = 64
DM = 512


def kernel(x, Wq, K_ext, V_ext, Wo):
    Kt = jnp.transpose(K_ext, (0, 2, 1, 3))
    Vt = jnp.transpose(V_ext, (0, 2, 1, 3))
    Wqt = jnp.transpose(Wq.reshape(DM, HQ, DH), (1, 0, 2)) * 0.125

    def body(x_ref, wqt_ref, k_ref, v_ref, wo_ref, out_ref,
             klbuf, vlbuf, krbuf, vrbuf, send_sems, recv_sems):
        my = lax.axis_index("i")
        left = jnp.maximum(my - 1, 0)
        right = jnp.minimum(my + 1, N_DEV - 1)

        @pl.when(my == 0)
        def _():
            vlbuf[...] = jnp.zeros((B, HQ, HALO, DH), jnp.float32)

        @pl.when(my == N_DEV - 1)
        def _():
            vrbuf[...] = jnp.zeros((B, HQ, HALO, DH), jnp.float32)

        rdma_r_k = pltpu.make_async_remote_copy(
            src_ref=k_ref.at[:, :, pl.ds(SQ - HALO, HALO)],
            dst_ref=klbuf,
            send_sem=send_sems.at[0], recv_sem=recv_sems.at[0],
            device_id=(right,), device_id_type=pltpu.DeviceIdType.MESH,
        )
        rdma_r_v = pltpu.make_async_remote_copy(
            src_ref=v_ref.at[:, :, pl.ds(SQ - HALO, HALO)],
            dst_ref=vlbuf,
            send_sem=send_sems.at[1], recv_sem=recv_sems.at[1],
            device_id=(right,), device_id_type=pltpu.DeviceIdType.MESH,
        )
        rdma_l_k = pltpu.make_async_remote_copy(
            src_ref=k_ref.at[:, :, pl.ds(0, HALO)],
            dst_ref=krbuf,
            send_sem=send_sems.at[2], recv_sem=recv_sems.at[2],
            device_id=(left,), device_id_type=pltpu.DeviceIdType.MESH,
        )
        rdma_l_v = pltpu.make_async_remote_copy(
            src_ref=v_ref.at[:, :, pl.ds(0, HALO)],
            dst_ref=vrbuf,
            send_sem=send_sems.at[3], recv_sem=recv_sems.at[3],
            device_id=(left,), device_id_type=pltpu.DeviceIdType.MESH,
        )

        @pl.when(my < N_DEV - 1)
        def _():
            rdma_r_k.start()
            rdma_r_v.start()

        @pl.when(my > 0)
        def _():
            rdma_l_k.start()
            rdma_l_v.start()

        r_a = lax.broadcasted_iota(jnp.int32, (SQ, SQ), 0)
        j_a = lax.broadcasted_iota(jnp.int32, (SQ, SQ), 1)
        mask_a = jnp.abs(j_a - r_a) <= HALO

        q = []
        ctx = []
        lsum = []
        for b in range(B):
            q.append([])
            ctx.append([])
            lsum.append([])
            for h in range(HQ):
                qh = jnp.dot(x_ref[b], wqt_ref[h],
                             preferred_element_type=jnp.float32)
                s = lax.dot_general(
                    qh, k_ref[b, h], (((1,), (1,)), ((), ())),
                    preferred_element_type=jnp.float32,
                )
                w = jnp.where(mask_a, jnp.exp(s), 0.0)
                ctx_h = jnp.dot(w, v_ref[b, h],
                                preferred_element_type=jnp.float32)
                q[b].append(qh)
                ctx[b].append(ctx_h)
                lsum[b].append(jnp.sum(w, axis=1, keepdims=True))

        @pl.when(my > 0)
        def _():
            rdma_r_k.wait_recv()
            rdma_r_v.wait_recv()

        @pl.when(my < N_DEV - 1)
        def _():
            rdma_l_k.wait_recv()
            rdma_l_v.wait_recv()

        @pl.when(my < N_DEV - 1)
        def _():
            rdma_r_k.wait_send()
            rdma_r_v.wait_send()

        @pl.when(my > 0)
        def _():
            rdma_l_k.wait_send()
            rdma_l_v.wait_send()

        r_h = lax.broadcasted_iota(jnp.int32, (SQ, HALO), 0)
        j_h = lax.broadcasted_iota(jnp.int32, (SQ, HALO), 1)
        mask_l = (j_h >= r_h) & (my > 0)
        mask_r = (j_h <= r_h - HALO) & (my < N_DEV - 1)

        for b in range(B):
            acc = jnp.zeros((SQ, DM), jnp.float32)
            for h in range(HQ):
                s_l = lax.dot_general(
                    q[b][h], klbuf[b, h], (((1,), (1,)), ((), ())),
                    preferred_element_type=jnp.float32,
                )
                s_r = lax.dot_general(
                    q[b][h], krbuf[b, h], (((1,), (1,)), ((), ())),
                    preferred_element_type=jnp.float32,
                )
                w_l = jnp.where(mask_l, jnp.exp(s_l), 0.0)
                w_r = jnp.where(mask_r, jnp.exp(s_r), 0.0)
                ctx_h = (
                    ctx[b][h]
                    + jnp.dot(w_l, vlbuf[b, h],
                              preferred_element_type=jnp.float32)
                    + jnp.dot(w_r, vrbuf[b, h],
                              preferred_element_type=jnp.float32)
                )
                l_h = (lsum[b][h]
                       + jnp.sum(w_l, axis=1, keepdims=True)
                       + jnp.sum(w_r, axis=1, keepdims=True))
                ctx_h = ctx_h / l_h
                acc += jnp.dot(ctx_h, wo_ref[h * DH:(h + 1) * DH, :],
                               preferred_element_type=jnp.float32)
            out_ref[b] = acc

    return pl.pallas_call(
        body,
        out_shape=jax.ShapeDtypeStruct(x.shape, jnp.float32),
        in_specs=[pl.BlockSpec(memory_space=pltpu.VMEM)] * 5,
        out_specs=pl.BlockSpec(memory_space=pltpu.VMEM),
        scratch_shapes=[
            pltpu.VMEM((B, HQ, HALO, DH), jnp.float32),
            pltpu.VMEM((B, HQ, HALO, DH), jnp.float32),
            pltpu.VMEM((B, HQ, HALO, DH), jnp.float32),
            pltpu.VMEM((B, HQ, HALO, DH), jnp.float32),
            pltpu.SemaphoreType.DMA((4,)),
            pltpu.SemaphoreType.DMA((4,)),
        ],
    )(x, Wqt, Kt, Vt, Wo)


# device time: 18429 ns/iter; 1.6169x vs baseline; 1.5596x over previous
import jax
import jax.numpy as jnp
from jax import lax
from jax.experimental import pallas as pl
from jax.experimental.pallas import tpu as pltpu

N_DEV = 8
B = 2
SQ = 256
HALO = 128
HQ = 4
DH = 64
DM = 512


def kernel(x, Wq, K_ext, V_ext, Wo):
    Kt = jnp.transpose(K_ext, (0, 2, 1, 3))
    Vt = jnp.transpose(V_ext, (0, 2, 1, 3))
    Wqt = jnp.transpose(Wq.reshape(DM, HQ, DH), (1, 0, 2)) * 0.125

    def body(x_ref, wqt_ref, k_ref, v_ref, wo_ref, out_ref,
             send_l, send_r, recv_l, recv_r, send_sems, recv_sems):
        my = lax.axis_index("i")
        left = jnp.maximum(my - 1, 0)
        right = jnp.minimum(my + 1, N_DEV - 1)

        barrier_sem = pltpu.get_barrier_semaphore()

        @pl.when(my > 0)
        def _():
            pl.semaphore_signal(barrier_sem, inc=1, device_id=(left,),
                                device_id_type=pltpu.DeviceIdType.MESH)

        @pl.when(my < N_DEV - 1)
        def _():
            pl.semaphore_signal(barrier_sem, inc=1, device_id=(right,),
                                device_id_type=pltpu.DeviceIdType.MESH)

        n_nbrs = (my > 0).astype(jnp.int32) + (my < N_DEV - 1).astype(jnp.int32)

        send_r[0] = k_ref[:, :, SQ - HALO:SQ].astype(jnp.bfloat16)
        send_r[1] = v_ref[:, :, SQ - HALO:SQ].astype(jnp.bfloat16)
        send_l[0] = k_ref[:, :, 0:HALO].astype(jnp.bfloat16)
        send_l[1] = v_ref[:, :, 0:HALO].astype(jnp.bfloat16)

        @pl.when(my == 0)
        def _():
            recv_l[1] = jnp.zeros((B, HQ, HALO, DH), jnp.bfloat16)

        @pl.when(my == N_DEV - 1)
        def _():
            recv_r[1] = jnp.zeros((B, HQ, HALO, DH), jnp.bfloat16)

        pl.semaphore_wait(barrier_sem, n_nbrs)

        rdma_r = pltpu.make_async_remote_copy(
            src_ref=send_r, dst_ref=recv_l,
            send_sem=send_sems.at[0], recv_sem=recv_sems.at[0],
            device_id=(right,), device_id_type=pltpu.DeviceIdType.MESH,
        )
        rdma_l = pltpu.make_async_remote_copy(
            src_ref=send_l, dst_ref=recv_r,
            send_sem=send_sems.at[1], recv_sem=recv_sems.at[1],
            device_id=(left,), device_id_type=pltpu.DeviceIdType.MESH,
        )

        @pl.when(my < N_DEV - 1)
        def _():
            rdma_r.start()

        @pl.when(my > 0)
        def _():
            rdma_l.start()

        r_a = lax.broadcasted_iota(jnp.int32, (SQ, SQ), 0)
        j_a = lax.broadcasted_iota(jnp.int32, (SQ, SQ), 1)
        mask_a = jnp.abs(j_a - r_a) <= HALO

        q = []
        ctx = []
        lsum = []
        for b in range(B):
            q.append([])
            ctx.append([])
            lsum.append([])
            for h in range(HQ):
                qh = jnp.dot(x_ref[b], wqt_ref[h],
                             preferred_element_type=jnp.float32)
                s = lax.dot_general(
                    qh, k_ref[b, h], (((1,), (1,)), ((), ())),
                    preferred_element_type=jnp.float32,
                )
                w = jnp.where(mask_a, jnp.exp(s), 0.0)
                ctx_h = jnp.dot(w, v_ref[b, h],
                                preferred_element_type=jnp.float32)
                q[b].append(qh)
                ctx[b].append(ctx_h)
                lsum[b].append(jnp.sum(w, axis=1, keepdims=True))

        @pl.when(my > 0)
        def _():
            rdma_r.wait_recv()

        @pl.when(my < N_DEV - 1)
        def _():
            rdma_l.wait_recv()

        @pl.when(my < N_DEV - 1)
        def _():
            rdma_r.wait_send()

        @pl.when(my > 0)
        def _():
            rdma_l.wait_send()

        r_h = lax.broadcasted_iota(jnp.int32, (SQ, HALO), 0)
        j_h = lax.broadcasted_iota(jnp.int32, (SQ, HALO), 1)
        mask_l = (j_h >= r_h) & (my > 0)
        mask_r = (j_h <= r_h - HALO) & (my < N_DEV - 1)

        for b in range(B):
            acc = jnp.zeros((SQ, DM), jnp.float32)
            for h in range(HQ):
                s_l = lax.dot_general(
                    q[b][h], recv_l[0, b, h].astype(jnp.float32),
                    (((1,), (1,)), ((), ())),
                    preferred_element_type=jnp.float32,
                )
                s_r = lax.dot_general(
                    q[b][h], recv_r[0, b, h].astype(jnp.float32),
                    (((1,), (1,)), ((), ())),
                    preferred_element_type=jnp.float32,
                )
                w_l = jnp.where(mask_l, jnp.exp(s_l), 0.0)
                w_r = jnp.where(mask_r, jnp.exp(s_r), 0.0)
                ctx_h = (
                    ctx[b][h]
                    + jnp.dot(w_l, recv_l[1, b, h].astype(jnp.float32),
                              preferred_element_type=jnp.float32)
                    + jnp.dot(w_r, recv_r[1, b, h].astype(jnp.float32),
                              preferred_element_type=jnp.float32)
                )
                l_h = (lsum[b][h]
                       + jnp.sum(w_l, axis=1, keepdims=True)
                       + jnp.sum(w_r, axis=1, keepdims=True))
                ctx_h = ctx_h / l_h
                acc += jnp.dot(ctx_h, wo_ref[h * DH:(h + 1) * DH, :],
                               preferred_element_type=jnp.float32)
            out_ref[b] = acc

    halo_shape = (2, B, HQ, HALO, DH)
    return pl.pallas_call(
        body,
        out_shape=jax.ShapeDtypeStruct(x.shape, jnp.float32),
        in_specs=[pl.BlockSpec(memory_space=pltpu.VMEM)] * 5,
        out_specs=pl.BlockSpec(memory_space=pltpu.VMEM),
        compiler_params=pltpu.CompilerParams(collective_id=0),
        scratch_shapes=[
            pltpu.VMEM(halo_shape, jnp.bfloat16),
            pltpu.VMEM(halo_shape, jnp.bfloat16),
            pltpu.VMEM(halo_shape, jnp.bfloat16),
            pltpu.VMEM(halo_shape, jnp.bfloat16),
            pltpu.SemaphoreType.DMA((2,)),
            pltpu.SemaphoreType.DMA((2,)),
        ],
    )(x, Wqt, Kt, Vt, Wo)
